# baseline (speedup 1.0000x reference)
import dataclasses
import functools

import jax
import jax.numpy as jnp
from jax import lax
from jax.experimental import pallas as pl
from jax.experimental.pallas import tpu as pltpu
from jax.experimental.pallas import tpu_sc as plsc

N = 10000
E = 320000
VOCAB = 1024
D = 128
B = 64

NC = 2
NS = 16
NW = NC * NS
LANES = 16

NP = 10240
CHUNK = 128
SUP = 27
NSUPER = 3
ET = NW * NSUPER * SUP * CHUNK
EWK = ET // NW
SUPE = SUP * CHUNK
ROWS_PER_SUB = NP // NS
ZROWS = 32
GROWS = NP // NW
SUBC = 80

_mesh = plsc.VectorSubcoreMesh(core_axis_name="c", subcore_axis_name="s")

_cp = pltpu.CompilerParams()
if "needs_layout_passes" in pltpu.CompilerParams.__dataclass_fields__:
    _cp = dataclasses.replace(_cp, needs_layout_passes=False)


def _wid():
    return lax.axis_index("s") * NC + lax.axis_index("c")


def _deg_body(cols_hbm, ew_hbm, degp_hbm, colbuf, ewbuf, degloc, sem):
    wid = _wid()
    zero16 = jnp.zeros((LANES,), jnp.float32)

    d1 = pltpu.async_copy(cols_hbm.at[pl.ds(wid * EWK, EWK)], colbuf, sem)
    d2 = pltpu.async_copy(ew_hbm.at[pl.ds(wid * EWK, EWK)], ewbuf, sem)

    @pl.loop(0, NP, step=LANES)
    def _(i):
        degloc[pl.ds(i, LANES)] = zero16

    d1.wait()
    d2.wait()

    @pl.loop(0, EWK, step=LANES)
    def _(g):
        cv = colbuf[pl.ds(g, LANES)]
        ev = ewbuf[pl.ds(g, LANES)]
        plsc.addupdate_scatter(degloc, [cv], ev)

    pltpu.async_copy(degloc, degp_hbm.at[wid], sem).wait()


@functools.partial(
    pl.kernel,
    out_type=jax.ShapeDtypeStruct((NW, NP), jnp.float32),
    mesh=_mesh,
    scratch_types=[
        pltpu.VMEM((EWK,), jnp.int32),
        pltpu.VMEM((EWK,), jnp.float32),
        pltpu.VMEM((NP,), jnp.float32),
        pltpu.SemaphoreType.DMA,
    ],
    compiler_params=_cp,
)
def _deg_kernel(cols_hbm, ew_hbm, degp_hbm, colbuf, ewbuf, degloc, sem):
    _deg_body(cols_hbm, ew_hbm, degp_hbm, colbuf, ewbuf, degloc, sem)


def _norm_body(rows_hbm, cols_hbm, ew_hbm, dinv_hbm, ni_hbm,
               norm_hbm, rows2_hbm,
               rowbuf, colbuf, ewbuf, dinvbuf, nibuf, sem):
    wid = _wid()
    d1 = pltpu.async_copy(rows_hbm.at[pl.ds(wid * EWK, EWK)], rowbuf, sem)
    d2 = pltpu.async_copy(cols_hbm.at[pl.ds(wid * EWK, EWK)], colbuf, sem)
    d3 = pltpu.async_copy(ew_hbm.at[pl.ds(wid * EWK, EWK)], ewbuf, sem)
    d4 = pltpu.async_copy(dinv_hbm, dinvbuf, sem)
    d5 = pltpu.async_copy(ni_hbm, nibuf, sem)
    d1.wait()
    d2.wait()
    d3.wait()
    d4.wait()
    d5.wait()

    @pl.loop(0, EWK, step=LANES)
    def _(g):
        sl = pl.ds(g, LANES)
        rv = rowbuf[sl]
        dr = plsc.load_gather(dinvbuf, [rv])
        dc = plsc.load_gather(dinvbuf, [colbuf[sl]])
        ewbuf[sl] = dr * ewbuf[sl] * dc
        rowbuf[sl] = plsc.load_gather(nibuf, [rv])

    d6 = pltpu.async_copy(ewbuf, norm_hbm.at[pl.ds(wid * EWK, EWK)], sem)
    d7 = pltpu.async_copy(rowbuf, rows2_hbm.at[pl.ds(wid * EWK, EWK)], sem)
    d6.wait()
    d7.wait()


@functools.partial(
    pl.kernel,
    out_type=(jax.ShapeDtypeStruct((ET,), jnp.float32),
              jax.ShapeDtypeStruct((ET,), jnp.int32)),
    mesh=_mesh,
    scratch_types=[
        pltpu.VMEM((EWK,), jnp.int32),
        pltpu.VMEM((EWK,), jnp.int32),
        pltpu.VMEM((EWK,), jnp.float32),
        pltpu.VMEM((NP,), jnp.float32),
        pltpu.VMEM((NP,), jnp.int32),
        pltpu.SemaphoreType.DMA,
    ],
    compiler_params=_cp,
)
def _norm_kernel(rows_hbm, cols_hbm, ew_hbm, dinv_hbm, ni_hbm,
                 norm_hbm, rows2_hbm,
                 rowbuf, colbuf, ewbuf, dinvbuf, nibuf, sem):
    _norm_body(rows_hbm, cols_hbm, ew_hbm, dinv_hbm, ni_hbm,
               norm_hbm, rows2_hbm,
               rowbuf, colbuf, ewbuf, dinvbuf, nibuf, sem)


def _mp_body(table_hbm, rows_hbm, cols_hbm, norm_hbm,
             outa_hbm, outb_hbm,
             rbuf, cbuf, nbuf, gbuf0, gbuf1, zbuf, acc,
             isem, gsem0, gsem1, ssem0, ssem1):
    cid = lax.axis_index("c")
    sid = lax.axis_index("s")
    wid = sid * NC + cid

    zero16 = jnp.zeros((LANES,), jnp.float32)
    gbufs = (gbuf0, gbuf1)
    gsems = (gsem0, gsem1)
    ssems = (ssem0, ssem1)

    @pl.loop(0, ZROWS)
    def _(r):
        for m in range(D // LANES):
            zbuf[r, pl.ds(m * LANES, LANES)] = zero16

    zdescs = [
        pltpu.async_copy(
            zbuf, acc.at[pl.ds(sid * ROWS_PER_SUB + k * ZROWS, ZROWS)], isem)
        for k in range(ROWS_PER_SUB // ZROWS)
    ]
    for d in zdescs:
        d.wait()
    plsc.subcore_barrier()

    @pl.loop(0, NSUPER)
    def _(s):
        p = wid * NSUPER + s
        pltpu.sync_copy(rows_hbm.at[p], rbuf)
        pltpu.sync_copy(cols_hbm.at[p], cbuf)
        pltpu.sync_copy(norm_hbm.at[pl.ds(wid * EWK + s * SUPE, SUPE)], nbuf)

        gdescs = [pltpu.async_copy(table_hbm.at[rbuf.at[0]], gbuf0, gsem0)]
        sdescs = [None, None]
        for k in range(SUP):
            b = k % 2
            gdescs[k].wait()
            if k < SUP - 1:
                nb = (k + 1) % 2
                if sdescs[nb] is not None:
                    sdescs[nb].wait()
                gdescs.append(
                    pltpu.async_copy(table_hbm.at[rbuf.at[k + 1]],
                                     gbufs[nb], gsems[nb]))

            @pl.loop(0, CHUNK, unroll=4)
            def _(e, _b=b, _k=k):
                nsplat = plsc.load_gather(
                    nbuf, [lax.broadcast(e + _k * CHUNK, (LANES,))])
                g = gbufs[_b]
                for m in range(D // LANES):
                    sl = pl.ds(m * LANES, LANES)
                    g[e, sl] = g[e, sl] * nsplat

            sdescs[b] = pltpu.async_copy(gbufs[b], acc.at[cbuf.at[k]],
                                         ssems[b], add=True)
        sdescs[0].wait()
        sdescs[1].wait()

    plsc.subcore_barrier()

    @pl.when(jnp.logical_and(cid == 0, sid < 15))
    def _():
        pltpu.sync_copy(acc.at[pl.ds(sid * ROWS_PER_SUB, ROWS_PER_SUB)],
                        outa_hbm.at[pl.ds(sid * ROWS_PER_SUB, ROWS_PER_SUB)])

    @pl.when(jnp.logical_and(cid == 0, sid == 15))
    def _():
        pltpu.sync_copy(acc.at[pl.ds(15 * ROWS_PER_SUB, N - 15 * ROWS_PER_SUB)],
                        outa_hbm.at[pl.ds(15 * ROWS_PER_SUB,
                                          N - 15 * ROWS_PER_SUB)])

    @pl.when(jnp.logical_and(cid == 1, sid < 15))
    def _():
        pltpu.sync_copy(acc.at[pl.ds(sid * ROWS_PER_SUB, ROWS_PER_SUB)],
                        outb_hbm.at[pl.ds(sid * ROWS_PER_SUB, ROWS_PER_SUB)])

    @pl.when(jnp.logical_and(cid == 1, sid == 15))
    def _():
        pltpu.sync_copy(acc.at[pl.ds(15 * ROWS_PER_SUB, N - 15 * ROWS_PER_SUB)],
                        outb_hbm.at[pl.ds(15 * ROWS_PER_SUB,
                                          N - 15 * ROWS_PER_SUB)])


@functools.partial(
    pl.kernel,
    out_type=(jax.ShapeDtypeStruct((N, D), jnp.float32),
              jax.ShapeDtypeStruct((N, D), jnp.float32)),
    mesh=_mesh,
    scratch_types=[
        pltpu.VMEM((SUP, CHUNK), jnp.int32),
        pltpu.VMEM((SUP, CHUNK), jnp.int32),
        pltpu.VMEM((SUPE,), jnp.float32),
        pltpu.VMEM((CHUNK, D), jnp.float32),
        pltpu.VMEM((CHUNK, D), jnp.float32),
        pltpu.VMEM((ZROWS, D), jnp.float32),
        pltpu.VMEM_SHARED((NP, D), jnp.float32),
        pltpu.SemaphoreType.DMA,
        pltpu.SemaphoreType.DMA,
        pltpu.SemaphoreType.DMA,
        pltpu.SemaphoreType.DMA,
        pltpu.SemaphoreType.DMA,
    ],
    compiler_params=_cp,
)
def _mp_kernel(table_hbm, rows_hbm, cols_hbm, norm_hbm,
               outa_hbm, outb_hbm,
               rbuf, cbuf, nbuf, gbuf0, gbuf1, zbuf, acc,
               isem, gsem0, gsem1, ssem0, ssem1):
    _mp_body(table_hbm, rows_hbm, cols_hbm, norm_hbm,
             outa_hbm, outb_hbm,
             rbuf, cbuf, nbuf, gbuf0, gbuf1, zbuf, acc,
             isem, gsem0, gsem1, ssem0, ssem1)


def _matmul_small_body(a_ref, w_ref, o_ref):
    o_ref[...] = jnp.dot(a_ref[...], w_ref[...],
                         preferred_element_type=jnp.float32)


def _tc_chordw(chord, W1):
    return pl.pallas_call(
        _matmul_small_body,
        out_shape=jax.ShapeDtypeStruct((VOCAB, D), jnp.float32),
    )(chord, W1)


def _dinv_body(degp_ref, o_ref):
    deg = degp_ref[pl.ds(0, NP // 128), :]
    for k in range(1, NW):
        deg = deg + degp_ref[pl.ds(k * (NP // 128), NP // 128), :]
    o_ref[...] = jnp.where(deg > 0, lax.rsqrt(deg), 0.0)


def _tc_dinv(degp2):
    return pl.pallas_call(
        _dinv_body,
        out_shape=jax.ShapeDtypeStruct((NP // 128, 128), jnp.float32),
    )(degp2)


ROW_BLK = 2000


def _layer1_body(acca_ref, accb_ref, b_ref, w2_ref, o_ref):
    h = acca_ref[...] + accb_ref[...] + b_ref[...]
    h = jnp.maximum(h, 0.0)
    o_ref[...] = jnp.dot(h, w2_ref[...], preferred_element_type=jnp.float32)


def _tc_layer1(acca, accb, b1, W2):
    blk = lambda i: (i, 0)
    return pl.pallas_call(
        _layer1_body,
        grid=(N // ROW_BLK,),
        in_specs=[
            pl.BlockSpec((ROW_BLK, D), blk),
            pl.BlockSpec((ROW_BLK, D), blk),
            pl.BlockSpec((1, D), lambda i: (0, 0)),
            pl.BlockSpec((D, D), lambda i: (0, 0)),
        ],
        out_specs=pl.BlockSpec((ROW_BLK, D), blk),
        out_shape=jax.ShapeDtypeStruct((N, D), jnp.float32),
    )(acca, accb, b1, W2)


def _layer2_pool_body(acca_ref, accb_ref, b_ref, bi_ref, o_ref,
                      sums_ref, cnt_ref):
    i = pl.program_id(0)

    @pl.when(i == 0)
    def _():
        sums_ref[...] = jnp.zeros_like(sums_ref)
        cnt_ref[...] = jnp.zeros_like(cnt_ref)

    h = acca_ref[...] + accb_ref[...] + b_ref[...]
    h = jnp.maximum(h, 0.0)
    oh = (bi_ref[...] == lax.broadcasted_iota(jnp.int32, (1, B), 1)
          ).astype(jnp.float32)
    dims = (((0,), (0,)), ((), ()))
    sums_ref[...] += lax.dot_general(oh, h, dims,
                                     preferred_element_type=jnp.float32)
    cnt_ref[...] += lax.dot_general(oh, jnp.ones((ROW_BLK, D), jnp.float32),
                                    dims, preferred_element_type=jnp.float32)

    @pl.when(i == pl.num_programs(0) - 1)
    def _():
        o_ref[...] = sums_ref[...] / jnp.maximum(cnt_ref[...], 1.0)


def _tc_layer2_pool(acca, accb, b2, bicol):
    blk = lambda i: (i, 0)
    return pl.pallas_call(
        _layer2_pool_body,
        grid=(N // ROW_BLK,),
        in_specs=[
            pl.BlockSpec((ROW_BLK, D), blk),
            pl.BlockSpec((ROW_BLK, D), blk),
            pl.BlockSpec((1, D), lambda i: (0, 0)),
            pl.BlockSpec((ROW_BLK, 1), blk),
        ],
        out_specs=pl.BlockSpec((B, D), lambda i: (0, 0)),
        out_shape=jax.ShapeDtypeStruct((B, D), jnp.float32),
        scratch_shapes=[
            pltpu.VMEM((B, D), jnp.float32),
            pltpu.VMEM((B, D), jnp.float32),
        ],
    )(acca, accb, b2, bicol)


def kernel(edge_index, edge_attr, node_indices, batch_index,
           chord_node_features, W1, b1, W2, b2):
    f32 = jnp.float32
    rows = edge_index[0].astype(jnp.int32)
    cols = edge_index[1].astype(jnp.int32)
    loop = jnp.arange(N, dtype=jnp.int32)
    padi = jnp.arange(ET - E - N, dtype=jnp.int32) * 7 % N
    rows_x = jnp.concatenate([rows, loop, padi])
    cols_x = jnp.concatenate([cols, loop, padi])
    ew_x = jnp.concatenate([edge_attr.astype(f32), jnp.ones((N,), f32),
                            jnp.zeros((ET - E - N,), f32)])
    rows3 = rows_x.reshape(NW * NSUPER, SUP, CHUNK)
    cols3 = cols_x.reshape(NW * NSUPER, SUP, CHUNK)
    ni = jnp.concatenate([node_indices.astype(jnp.int32),
                          jnp.zeros((NP - N,), jnp.int32)])

    chordw1 = _tc_chordw(chord_node_features.astype(f32), W1.astype(f32))
    degp = _deg_kernel(cols_x, ew_x)
    dinv = _tc_dinv(degp.reshape(NW * (NP // 128), 128)).reshape(NP)
    norm_x, rows2 = _norm_kernel(rows_x, cols_x, ew_x, dinv, ni)
    rows2_3 = rows2.reshape(NW * NSUPER, SUP, CHUNK)

    acc1a, acc1b = _mp_kernel(chordw1, rows2_3, cols3, norm_x)
    t2 = _tc_layer1(acc1a, acc1b, b1.astype(f32).reshape(1, D),
                    W2.astype(f32))

    acc2a, acc2b = _mp_kernel(t2, rows3, cols3, norm_x)
    bicol = batch_index.astype(jnp.int32).reshape(N, 1)
    return _tc_layer2_pool(acc2a, acc2b, b2.astype(f32).reshape(1, D), bicol)


# speedup vs baseline: 1.1473x; 1.1473x over previous
import dataclasses
import functools

import jax
import jax.numpy as jnp
from jax import lax
from jax.experimental import pallas as pl
from jax.experimental.pallas import tpu as pltpu
from jax.experimental.pallas import tpu_sc as plsc

N = 10000
E = 320000
VOCAB = 1024
D = 128
B = 64

NC = 2
NS = 16
NW = NC * NS
LANES = 16

NP = 10240
CHUNK = 128
SUP = 27
NSUPER = 3
ET = NW * NSUPER * SUP * CHUNK
EWK = ET // NW
SUPE = SUP * CHUNK
ROWS_PER_SUB = NP // NS
ZROWS = 32
GROWS = NP // NW
SUBC = 80

_mesh = plsc.VectorSubcoreMesh(core_axis_name="c", subcore_axis_name="s")

_cp = pltpu.CompilerParams()
if "needs_layout_passes" in pltpu.CompilerParams.__dataclass_fields__:
    _cp = dataclasses.replace(_cp, needs_layout_passes=False)


def _wid():
    return lax.axis_index("s") * NC + lax.axis_index("c")


def _deg_body(cols_hbm, ew_hbm, degp_hbm, colbuf, ewbuf, degloc, sem):
    wid = _wid()
    zero16 = jnp.zeros((LANES,), jnp.float32)

    d1 = pltpu.async_copy(cols_hbm.at[pl.ds(wid * EWK, EWK)], colbuf, sem)
    d2 = pltpu.async_copy(ew_hbm.at[pl.ds(wid * EWK, EWK)], ewbuf, sem)

    @pl.loop(0, NP, step=LANES)
    def _(i):
        degloc[pl.ds(i, LANES)] = zero16

    d1.wait()
    d2.wait()

    @pl.loop(0, EWK, step=LANES)
    def _(g):
        cv = colbuf[pl.ds(g, LANES)]
        ev = ewbuf[pl.ds(g, LANES)]
        plsc.addupdate_scatter(degloc, [cv], ev)

    pltpu.async_copy(degloc, degp_hbm.at[wid], sem).wait()


@functools.partial(
    pl.kernel,
    out_type=jax.ShapeDtypeStruct((NW, NP), jnp.float32),
    mesh=_mesh,
    scratch_types=[
        pltpu.VMEM((EWK,), jnp.int32),
        pltpu.VMEM((EWK,), jnp.float32),
        pltpu.VMEM((NP,), jnp.float32),
        pltpu.SemaphoreType.DMA,
    ],
    compiler_params=_cp,
)
def _deg_kernel(cols_hbm, ew_hbm, degp_hbm, colbuf, ewbuf, degloc, sem):
    _deg_body(cols_hbm, ew_hbm, degp_hbm, colbuf, ewbuf, degloc, sem)


def _norm_body(rows_hbm, cols_hbm, ew_hbm, dinv_hbm, ni_hbm,
               norm_hbm, rows2_hbm,
               rowbuf, colbuf, ewbuf, dinvbuf, nibuf, sem):
    wid = _wid()
    d1 = pltpu.async_copy(rows_hbm.at[pl.ds(wid * EWK, EWK)], rowbuf, sem)
    d2 = pltpu.async_copy(cols_hbm.at[pl.ds(wid * EWK, EWK)], colbuf, sem)
    d3 = pltpu.async_copy(ew_hbm.at[pl.ds(wid * EWK, EWK)], ewbuf, sem)
    d4 = pltpu.async_copy(dinv_hbm, dinvbuf, sem)
    d5 = pltpu.async_copy(ni_hbm, nibuf, sem)
    d1.wait()
    d2.wait()
    d3.wait()
    d4.wait()
    d5.wait()

    @pl.loop(0, EWK, step=LANES)
    def _(g):
        sl = pl.ds(g, LANES)
        rv = rowbuf[sl]
        dr = plsc.load_gather(dinvbuf, [rv])
        dc = plsc.load_gather(dinvbuf, [colbuf[sl]])
        ewbuf[sl] = dr * ewbuf[sl] * dc
        rowbuf[sl] = plsc.load_gather(nibuf, [rv])

    d6 = pltpu.async_copy(ewbuf, norm_hbm.at[pl.ds(wid * EWK, EWK)], sem)
    d7 = pltpu.async_copy(rowbuf, rows2_hbm.at[pl.ds(wid * EWK, EWK)], sem)
    d6.wait()
    d7.wait()


@functools.partial(
    pl.kernel,
    out_type=(jax.ShapeDtypeStruct((ET,), jnp.float32),
              jax.ShapeDtypeStruct((ET,), jnp.int32)),
    mesh=_mesh,
    scratch_types=[
        pltpu.VMEM((EWK,), jnp.int32),
        pltpu.VMEM((EWK,), jnp.int32),
        pltpu.VMEM((EWK,), jnp.float32),
        pltpu.VMEM((NP,), jnp.float32),
        pltpu.VMEM((NP,), jnp.int32),
        pltpu.SemaphoreType.DMA,
    ],
    compiler_params=_cp,
)
def _norm_kernel(rows_hbm, cols_hbm, ew_hbm, dinv_hbm, ni_hbm,
                 norm_hbm, rows2_hbm,
                 rowbuf, colbuf, ewbuf, dinvbuf, nibuf, sem):
    _norm_body(rows_hbm, cols_hbm, ew_hbm, dinv_hbm, ni_hbm,
               norm_hbm, rows2_hbm,
               rowbuf, colbuf, ewbuf, dinvbuf, nibuf, sem)


def _mp_body(table_hbm, rows_hbm, cols_hbm, norm_hbm,
             outa_hbm, outb_hbm,
             rbuf, cbuf, nbuf, gbuf0, gbuf1, zbuf, acc,
             isem, gsem0, gsem1, ssem0, ssem1):
    cid = lax.axis_index("c")
    sid = lax.axis_index("s")
    wid = sid * NC + cid

    zero16 = jnp.zeros((LANES,), jnp.float32)
    gbufs = (gbuf0, gbuf1)
    gsems = (gsem0, gsem1)
    ssems = (ssem0, ssem1)

    @pl.loop(0, ZROWS)
    def _(r):
        for m in range(D // LANES):
            zbuf[r, pl.ds(m * LANES, LANES)] = zero16

    zdescs = [
        pltpu.async_copy(
            zbuf, acc.at[pl.ds(sid * ROWS_PER_SUB + k * ZROWS, ZROWS)], isem)
        for k in range(ROWS_PER_SUB // ZROWS)
    ]
    for d in zdescs:
        d.wait()
    plsc.subcore_barrier()

    @pl.loop(0, NSUPER)
    def _(s):
        p = wid * NSUPER + s
        pltpu.sync_copy(rows_hbm.at[p], rbuf)
        pltpu.sync_copy(cols_hbm.at[p], cbuf)
        pltpu.sync_copy(norm_hbm.at[pl.ds(wid * EWK + s * SUPE, SUPE)], nbuf)

        gdescs = [pltpu.async_copy(table_hbm.at[rbuf.at[0]], gbuf0, gsem0)]
        sdescs = [None, None]
        for k in range(SUP):
            b = k % 2
            gdescs[k].wait()
            if k < SUP - 1:
                nb = (k + 1) % 2
                if sdescs[nb] is not None:
                    sdescs[nb].wait()
                gdescs.append(
                    pltpu.async_copy(table_hbm.at[rbuf.at[k + 1]],
                                     gbufs[nb], gsems[nb]))

            @plsc.parallel_loop(0, CHUNK, 1, unroll=2)
            def _(e, _b=b, _k=k):
                nsplat = plsc.load_gather(
                    nbuf, [lax.broadcast(e + _k * CHUNK, (LANES,))])
                g = gbufs[_b]
                for m in range(D // LANES):
                    sl = pl.ds(m * LANES, LANES)
                    g[e, sl] = g[e, sl] * nsplat

            sdescs[b] = pltpu.async_copy(gbufs[b], acc.at[cbuf.at[k]],
                                         ssems[b], add=True)
        sdescs[0].wait()
        sdescs[1].wait()

    plsc.subcore_barrier()

    @pl.when(jnp.logical_and(cid == 0, sid < 15))
    def _():
        pltpu.sync_copy(acc.at[pl.ds(sid * ROWS_PER_SUB, ROWS_PER_SUB)],
                        outa_hbm.at[pl.ds(sid * ROWS_PER_SUB, ROWS_PER_SUB)])

    @pl.when(jnp.logical_and(cid == 0, sid == 15))
    def _():
        pltpu.sync_copy(acc.at[pl.ds(15 * ROWS_PER_SUB, N - 15 * ROWS_PER_SUB)],
                        outa_hbm.at[pl.ds(15 * ROWS_PER_SUB,
                                          N - 15 * ROWS_PER_SUB)])

    @pl.when(jnp.logical_and(cid == 1, sid < 15))
    def _():
        pltpu.sync_copy(acc.at[pl.ds(sid * ROWS_PER_SUB, ROWS_PER_SUB)],
                        outb_hbm.at[pl.ds(sid * ROWS_PER_SUB, ROWS_PER_SUB)])

    @pl.when(jnp.logical_and(cid == 1, sid == 15))
    def _():
        pltpu.sync_copy(acc.at[pl.ds(15 * ROWS_PER_SUB, N - 15 * ROWS_PER_SUB)],
                        outb_hbm.at[pl.ds(15 * ROWS_PER_SUB,
                                          N - 15 * ROWS_PER_SUB)])


@functools.partial(
    pl.kernel,
    out_type=(jax.ShapeDtypeStruct((N, D), jnp.float32),
              jax.ShapeDtypeStruct((N, D), jnp.float32)),
    mesh=_mesh,
    scratch_types=[
        pltpu.VMEM((SUP, CHUNK), jnp.int32),
        pltpu.VMEM((SUP, CHUNK), jnp.int32),
        pltpu.VMEM((SUPE,), jnp.float32),
        pltpu.VMEM((CHUNK, D), jnp.float32),
        pltpu.VMEM((CHUNK, D), jnp.float32),
        pltpu.VMEM((ZROWS, D), jnp.float32),
        pltpu.VMEM_SHARED((NP, D), jnp.float32),
        pltpu.SemaphoreType.DMA,
        pltpu.SemaphoreType.DMA,
        pltpu.SemaphoreType.DMA,
        pltpu.SemaphoreType.DMA,
        pltpu.SemaphoreType.DMA,
    ],
    compiler_params=_cp,
)
def _mp_kernel(table_hbm, rows_hbm, cols_hbm, norm_hbm,
               outa_hbm, outb_hbm,
               rbuf, cbuf, nbuf, gbuf0, gbuf1, zbuf, acc,
               isem, gsem0, gsem1, ssem0, ssem1):
    _mp_body(table_hbm, rows_hbm, cols_hbm, norm_hbm,
             outa_hbm, outb_hbm,
             rbuf, cbuf, nbuf, gbuf0, gbuf1, zbuf, acc,
             isem, gsem0, gsem1, ssem0, ssem1)


def _matmul_small_body(a_ref, w_ref, o_ref):
    o_ref[...] = jnp.dot(a_ref[...], w_ref[...],
                         preferred_element_type=jnp.float32)


def _tc_chordw(chord, W1):
    return pl.pallas_call(
        _matmul_small_body,
        out_shape=jax.ShapeDtypeStruct((VOCAB, D), jnp.float32),
    )(chord, W1)


def _dinv_body(degp_ref, o_ref):
    deg = degp_ref[pl.ds(0, NP // 128), :]
    for k in range(1, NW):
        deg = deg + degp_ref[pl.ds(k * (NP // 128), NP // 128), :]
    o_ref[...] = jnp.where(deg > 0, lax.rsqrt(deg), 0.0)


def _tc_dinv(degp2):
    return pl.pallas_call(
        _dinv_body,
        out_shape=jax.ShapeDtypeStruct((NP // 128, 128), jnp.float32),
    )(degp2)


ROW_BLK = 2000


def _layer1_body(acca_ref, accb_ref, b_ref, w2_ref, o_ref):
    h = acca_ref[...] + accb_ref[...] + b_ref[...]
    h = jnp.maximum(h, 0.0)
    o_ref[...] = jnp.dot(h, w2_ref[...], preferred_element_type=jnp.float32)


def _tc_layer1(acca, accb, b1, W2):
    blk = lambda i: (i, 0)
    return pl.pallas_call(
        _layer1_body,
        grid=(N // ROW_BLK,),
        in_specs=[
            pl.BlockSpec((ROW_BLK, D), blk),
            pl.BlockSpec((ROW_BLK, D), blk),
            pl.BlockSpec((1, D), lambda i: (0, 0)),
            pl.BlockSpec((D, D), lambda i: (0, 0)),
        ],
        out_specs=pl.BlockSpec((ROW_BLK, D), blk),
        out_shape=jax.ShapeDtypeStruct((N, D), jnp.float32),
    )(acca, accb, b1, W2)


def _layer2_pool_body(acca_ref, accb_ref, b_ref, bi_ref, o_ref,
                      sums_ref, cnt_ref):
    i = pl.program_id(0)

    @pl.when(i == 0)
    def _():
        sums_ref[...] = jnp.zeros_like(sums_ref)
        cnt_ref[...] = jnp.zeros_like(cnt_ref)

    h = acca_ref[...] + accb_ref[...] + b_ref[...]
    h = jnp.maximum(h, 0.0)
    oh = (bi_ref[...] == lax.broadcasted_iota(jnp.int32, (1, B), 1)
          ).astype(jnp.float32)
    dims = (((0,), (0,)), ((), ()))
    sums_ref[...] += lax.dot_general(oh, h, dims,
                                     preferred_element_type=jnp.float32)
    cnt_ref[...] += lax.dot_general(oh, jnp.ones((ROW_BLK, D), jnp.float32),
                                    dims, preferred_element_type=jnp.float32)

    @pl.when(i == pl.num_programs(0) - 1)
    def _():
        o_ref[...] = sums_ref[...] / jnp.maximum(cnt_ref[...], 1.0)


def _tc_layer2_pool(acca, accb, b2, bicol):
    blk = lambda i: (i, 0)
    return pl.pallas_call(
        _layer2_pool_body,
        grid=(N // ROW_BLK,),
        in_specs=[
            pl.BlockSpec((ROW_BLK, D), blk),
            pl.BlockSpec((ROW_BLK, D), blk),
            pl.BlockSpec((1, D), lambda i: (0, 0)),
            pl.BlockSpec((ROW_BLK, 1), blk),
        ],
        out_specs=pl.BlockSpec((B, D), lambda i: (0, 0)),
        out_shape=jax.ShapeDtypeStruct((B, D), jnp.float32),
        scratch_shapes=[
            pltpu.VMEM((B, D), jnp.float32),
            pltpu.VMEM((B, D), jnp.float32),
        ],
    )(acca, accb, b2, bicol)


def kernel(edge_index, edge_attr, node_indices, batch_index,
           chord_node_features, W1, b1, W2, b2):
    f32 = jnp.float32
    rows = edge_index[0].astype(jnp.int32)
    cols = edge_index[1].astype(jnp.int32)
    loop = jnp.arange(N, dtype=jnp.int32)
    padi = jnp.arange(ET - E - N, dtype=jnp.int32) * 7 % N
    rows_x = jnp.concatenate([rows, loop, padi])
    cols_x = jnp.concatenate([cols, loop, padi])
    ew_x = jnp.concatenate([edge_attr.astype(f32), jnp.ones((N,), f32),
                            jnp.zeros((ET - E - N,), f32)])
    rows3 = rows_x.reshape(NW * NSUPER, SUP, CHUNK)
    cols3 = cols_x.reshape(NW * NSUPER, SUP, CHUNK)
    ni = jnp.concatenate([node_indices.astype(jnp.int32),
                          jnp.zeros((NP - N,), jnp.int32)])

    chordw1 = _tc_chordw(chord_node_features.astype(f32), W1.astype(f32))
    degp = _deg_kernel(cols_x, ew_x)
    dinv = _tc_dinv(degp.reshape(NW * (NP // 128), 128)).reshape(NP)
    norm_x, rows2 = _norm_kernel(rows_x, cols_x, ew_x, dinv, ni)
    rows2_3 = rows2.reshape(NW * NSUPER, SUP, CHUNK)

    acc1a, acc1b = _mp_kernel(chordw1, rows2_3, cols3, norm_x)
    t2 = _tc_layer1(acc1a, acc1b, b1.astype(f32).reshape(1, D),
                    W2.astype(f32))

    acc2a, acc2b = _mp_kernel(t2, rows3, cols3, norm_x)
    bicol = batch_index.astype(jnp.int32).reshape(N, 1)
    return _tc_layer2_pool(acc2a, acc2b, b2.astype(f32).reshape(1, D), bicol)
